# baseline (speedup 1.0000x reference)
import jax
import jax.numpy as jnp
from jax import lax
from jax.experimental import pallas as pl


_MIX = 0x45D9F3B
_FOLD = 48577
_LOW20 = 0xFFFFF
_NB = 999999


def _bucket(v):
    h = v ^ lax.shift_right_logical(v, 16)
    h = h * _MIX
    h = h ^ lax.shift_right_logical(h, 16)
    h = h * _MIX
    h = h ^ lax.shift_right_logical(h, 16)
    t = lax.shift_right_logical(h, 20) * _FOLD + (h & _LOW20)
    t = lax.shift_right_logical(t, 20) * _FOLD + (t & _LOW20)
    t = lax.shift_right_logical(t, 20) * _FOLD + (t & _LOW20)
    t = jnp.where(t >= _NB, t - _NB, t)
    return jnp.where(v == 0, 0, t + 1)


def _tc_body(x_ref, o_ref):
    o_ref[...] = _bucket(x_ref[...])


def kernel(x):
    xf = x.reshape(3200, 1024)
    out = pl.pallas_call(
        _tc_body,
        out_shape=jax.ShapeDtypeStruct((3200, 1024), jnp.int32),
        grid=(16,),
        in_specs=[pl.BlockSpec((200, 1024), lambda i: (i, 0))],
        out_specs=pl.BlockSpec((200, 1024), lambda i: (i, 0)),
    )(xf)
    return out.reshape(16384, 200)


# speedup vs baseline: 2.1598x; 2.1598x over previous
import jax
import jax.numpy as jnp
from jax import lax
from jax.experimental import pallas as pl


_MIX = 0x45D9F3B
_FOLD = 48577
_LOW20 = 0xFFFFF
_NB = 999999


def _bucket(v):
    h = v ^ lax.shift_right_logical(v, 16)
    h = h * _MIX
    h = h ^ lax.shift_right_logical(h, 16)
    h = h * _MIX
    h = h ^ lax.shift_right_logical(h, 16)
    t = lax.shift_right_logical(h, 20) * _FOLD + (h & _LOW20)
    t = lax.shift_right_logical(t, 20) * _FOLD + (t & _LOW20)
    t = lax.shift_right_logical(t, 20) * _FOLD + (t & _LOW20)
    t = jnp.where(t >= _NB, t - _NB, t)
    return jnp.where(v == 0, 0, t + 1)


def _tc_body(x_ref, o_ref):
    o_ref[...] = _bucket(x_ref[...])


def kernel(x):
    return pl.pallas_call(
        _tc_body,
        out_shape=jax.ShapeDtypeStruct((16384, 200), jnp.int32),
        grid=(16,),
        in_specs=[pl.BlockSpec((1024, 200), lambda i: (i, 0))],
        out_specs=pl.BlockSpec((1024, 200), lambda i: (i, 0)),
    )(x)


# speedup vs baseline: 2.2134x; 1.0248x over previous
import jax
import jax.numpy as jnp
from jax import lax
from jax.experimental import pallas as pl


_MIX = 0x45D9F3B
_NB = 999999
_MAGIC = 1125901033


def _bucket(v):
    u = v.astype(jnp.uint32)
    h = u ^ (u >> 16)
    h = h * jnp.uint32(_MIX)
    h = h ^ (h >> 16)
    h = h * jnp.uint32(_MIX)
    h = h ^ (h >> 16)
    t = (h % jnp.uint32(_NB)).astype(jnp.int32)
    return jnp.where(v == 0, 0, t + 1)


def _tc_body(x_ref, o_ref):
    o_ref[...] = _bucket(x_ref[...])


def kernel(x):
    return pl.pallas_call(
        _tc_body,
        out_shape=jax.ShapeDtypeStruct((16384, 200), jnp.int32),
        grid=(16,),
        in_specs=[pl.BlockSpec((1024, 200), lambda i: (i, 0))],
        out_specs=pl.BlockSpec((1024, 200), lambda i: (i, 0)),
    )(x)
